# baseline (speedup 1.0000x reference)
import functools

import jax
import jax.numpy as jnp
import numpy as np
from jax import lax
from jax.experimental import pallas as pl
from jax.experimental.pallas import tpu as pltpu
from jax.experimental.pallas import tpu_sc as plsc

_KEEP_RATIO = 0.5
_ROWS1 = 1024
_ROWS2 = 1024
_L = 16

_I32_MIN = np.int32(-(2 ** 31))
_I32_MAX = np.int32(2 ** 31 - 1)


def _sc_select_kernel(keep, D, imp_hbm, out_hbm, imp_v, key_v, out_v):
    nch = D // _L

    @pl.when(jnp.logical_and(lax.axis_index("c") == 0,
                             lax.axis_index("s") == 0))
    def _():
        pltpu.sync_copy(imp_hbm, imp_v)

        for c in range(nch):
            s = lax.bitcast_convert_type(imp_v[pl.ds(c * _L, _L)], jnp.int32)
            key_v[pl.ds(c * _L, _L)] = jnp.where(s >= 0, s, _I32_MIN - s)

        lanes = lax.iota(jnp.int32, _L)

        def cnt_gt(tv):
            acc = jnp.zeros((_L,), jnp.int32)
            for c in range(nch):
                kc = key_v[pl.ds(c * _L, _L)]
                acc = acc + plsc.all_reduce_population_count(kc > tv)
            return acc

        def bs1(_, carry):
            lo, hi = carry
            mid = (lo >> 1) + (hi >> 1) + (lo & hi & 1)
            big = cnt_gt(mid) >= keep
            return (jnp.where(big, mid + 1, lo), jnp.where(big, hi, mid))

        T, _ = lax.fori_loop(0, 32, bs1,
                             (jnp.full((_L,), _I32_MIN, jnp.int32),
                              jnp.full((_L,), _I32_MAX, jnp.int32)))
        r = keep - cnt_gt(T)

        def cnt_eq_below(jv):
            acc = jnp.zeros((_L,), jnp.int32)
            for c in range(nch):
                kc = key_v[pl.ds(c * _L, _L)]
                idx = lanes + (c * _L)
                acc = acc + plsc.all_reduce_population_count(
                    jnp.logical_and(kc == T, idx < jv))
            return acc

        def bs2(_, carry):
            lo, hi = carry
            mid = (lo + hi) >> 1
            ge = cnt_eq_below(mid) >= r
            return (jnp.where(ge, lo, mid + 1), jnp.where(ge, mid, hi))

        J, _ = lax.fori_loop(0, 12, bs2,
                             (jnp.zeros((_L,), jnp.int32),
                              jnp.full((_L,), D, jnp.int32)))

        for c in range(nch):
            kc = key_v[pl.ds(c * _L, _L)]
            idx = lanes + (c * _L)
            sel = jnp.logical_or(
                kc > T, jnp.logical_and(kc == T, idx < J))
            out_v[pl.ds(c * _L, _L)] = jnp.where(sel, 1.0, 0.0)

        pltpu.sync_copy(out_v, out_hbm)


def _make_sc_select(keep, D):
    mesh = plsc.VectorSubcoreMesh(core_axis_name="c", subcore_axis_name="s")
    return pl.kernel(
        functools.partial(_sc_select_kernel, keep, D),
        mesh=mesh,
        out_type=jax.ShapeDtypeStruct((D,), jnp.float32),
        scratch_types=[
            pltpu.VMEM((D,), jnp.float32),
            pltpu.VMEM((D,), jnp.int32),
            pltpu.VMEM((D,), jnp.float32),
        ],
        compiler_params=pltpu.CompilerParams(needs_layout_passes=False),
    )


def _tc_kernel(nsteps1, mask_ref, x1_ref, x2_ref, coef_ref,
               o_ref, acc_ref, ab_ref, loss_ref):
    i = pl.program_id(0)
    deg = ab_ref.shape[0]

    @pl.when(i == 0)
    def _init():
        m = mask_ref[...] > 0.5
        for k in range(deg):
            ab_ref[k:k + 1, :] = jnp.where(
                m, coef_ref[0, k], 1.0 if k == 0 else 0.0)
        acc_ref[...] = jnp.zeros_like(acc_ref)

    @pl.when(i < nsteps1)
    def _phase1():
        xb = x1_ref[...]
        acc_ref[...] = acc_ref[...] + jnp.sum(xb * xb, axis=0, keepdims=True)

    @pl.when(i == nsteps1 - 1)
    def _loss():
        loss_ref[0, 0] = 1e-6 * jnp.sqrt(
            jnp.sum(acc_ref[...] * (1.0 - mask_ref[...])))

    @pl.when(i >= nsteps1)
    def _phase2():
        x = x2_ref[...]
        p = ab_ref[deg - 1:deg, :] * x
        for k in range(deg - 2, -1, -1):
            p = (p + ab_ref[k:k + 1, :]) * x
        o_ref[...] = p + loss_ref[0, 0]


def kernel(x, coeffs, importance):
    B, T, D = x.shape
    keep = max(1, int(D * _KEEP_RATIO))
    deg = coeffs.shape[0]
    n = B * T
    nsteps1 = n // _ROWS1
    nsteps2 = n // _ROWS2
    xf = x.reshape(n, D)

    mask = _make_sc_select(keep, D)(importance).reshape(1, D)

    y = pl.pallas_call(
        functools.partial(_tc_kernel, nsteps1),
        grid=(nsteps1 + nsteps2,),
        in_specs=[
            pl.BlockSpec((1, D), lambda i: (0, 0)),
            pl.BlockSpec((_ROWS1, D), lambda i: (jnp.minimum(i, nsteps1 - 1), 0)),
            pl.BlockSpec((_ROWS2, D), lambda i: (jnp.maximum(i - nsteps1, 0), 0)),
            pl.BlockSpec(memory_space=pltpu.SMEM),
        ],
        out_specs=pl.BlockSpec((_ROWS2, D), lambda i: (jnp.maximum(i - nsteps1, 0), 0)),
        out_shape=jax.ShapeDtypeStruct((n, D), jnp.float32),
        scratch_shapes=[
            pltpu.VMEM((1, D), jnp.float32),
            pltpu.VMEM((deg, D), jnp.float32),
            pltpu.SMEM((1, 1), jnp.float32),
        ],
    )(mask, xf, xf, coeffs.reshape(1, deg))

    return y.reshape(B, T, D)


# speedup vs baseline: 1.2797x; 1.2797x over previous
import functools

import jax
import jax.numpy as jnp
import numpy as np
from jax import lax
from jax.experimental import pallas as pl
from jax.experimental.pallas import tpu as pltpu
from jax.experimental.pallas import tpu_sc as plsc

_KEEP_RATIO = 0.5
_ROWS1 = 1024
_ROWS2 = 1024
_L = 16

_I32_MIN = np.int32(-(2 ** 31))
_I32_MAX = np.int32(2 ** 31 - 1)


def _sc_select_kernel(keep, D, imp_hbm, out_hbm, imp_v, key_v, out_v):
    nch = D // _L

    @pl.when(jnp.logical_and(lax.axis_index("c") == 0,
                             lax.axis_index("s") == 0))
    def _():
        pltpu.sync_copy(imp_hbm, imp_v)

        for c in range(nch):
            s = lax.bitcast_convert_type(imp_v[pl.ds(c * _L, _L)], jnp.int32)
            key_v[pl.ds(c * _L, _L)] = jnp.where(s >= 0, s, _I32_MIN - s)

        lanes = lax.iota(jnp.int32, _L)

        def cnt_gt(tv):
            acc = jnp.zeros((_L,), jnp.int32)
            for c in range(nch):
                kc = key_v[pl.ds(c * _L, _L)]
                acc = acc + plsc.all_reduce_population_count(kc > tv)
            return acc

        def bs1(_, carry):
            lo, hi = carry
            mid = (lo >> 1) + (hi >> 1) + (lo & hi & 1)
            big = cnt_gt(mid) >= keep
            return (jnp.where(big, mid + 1, lo), jnp.where(big, hi, mid))

        T, _ = lax.fori_loop(0, 32, bs1,
                             (jnp.full((_L,), _I32_MIN, jnp.int32),
                              jnp.full((_L,), _I32_MAX, jnp.int32)))
        r = keep - cnt_gt(T)

        def cnt_eq_below(jv):
            acc = jnp.zeros((_L,), jnp.int32)
            for c in range(nch):
                kc = key_v[pl.ds(c * _L, _L)]
                idx = lanes + (c * _L)
                acc = acc + plsc.all_reduce_population_count(
                    jnp.logical_and(kc == T, idx < jv))
            return acc

        def bs2(_, carry):
            lo, hi = carry
            mid = (lo + hi) >> 1
            ge = cnt_eq_below(mid) >= r
            return (jnp.where(ge, lo, mid + 1), jnp.where(ge, mid, hi))

        J, _ = lax.fori_loop(0, 12, bs2,
                             (jnp.zeros((_L,), jnp.int32),
                              jnp.full((_L,), D, jnp.int32)))

        for c in range(nch):
            kc = key_v[pl.ds(c * _L, _L)]
            idx = lanes + (c * _L)
            sel = jnp.logical_or(
                kc > T, jnp.logical_and(kc == T, idx < J))
            out_v[pl.ds(c * _L, _L)] = jnp.where(sel, 1.0, 0.0)

        pltpu.sync_copy(out_v, out_hbm)


def _make_sc_select(keep, D):
    mesh = plsc.VectorSubcoreMesh(core_axis_name="c", subcore_axis_name="s")
    return pl.kernel(
        functools.partial(_sc_select_kernel, keep, D),
        mesh=mesh,
        out_type=jax.ShapeDtypeStruct((D,), jnp.float32),
        scratch_types=[
            pltpu.VMEM((D,), jnp.float32),
            pltpu.VMEM((D,), jnp.int32),
            pltpu.VMEM((D,), jnp.float32),
        ],
        compiler_params=pltpu.CompilerParams(needs_layout_passes=False),
    )


def _tc_sumsq_kernel(x_ref, o_ref):
    @pl.when(pl.program_id(0) == 0)
    def _init():
        o_ref[...] = jnp.zeros_like(o_ref)

    xb = x_ref[...]
    o_ref[...] = o_ref[...] + jnp.sum(xb * xb, axis=0, keepdims=True)


def _tc_out_kernel(mask_ref, colsum_ref, x_ref, coef_ref, o_ref,
                   ab_ref, loss_ref):
    i = pl.program_id(0)
    deg = ab_ref.shape[0]

    @pl.when(i == 0)
    def _init():
        m = mask_ref[...] > 0.5
        for k in range(deg):
            ab_ref[k:k + 1, :] = jnp.where(
                m, coef_ref[0, k], 1.0 if k == 0 else 0.0)
        loss_ref[0, 0] = 1e-6 * jnp.sqrt(
            jnp.sum(colsum_ref[...] * (1.0 - mask_ref[...])))

    x = x_ref[...]
    p = ab_ref[deg - 1:deg, :] * x
    for k in range(deg - 2, -1, -1):
        p = (p + ab_ref[k:k + 1, :]) * x
    o_ref[...] = p + loss_ref[0, 0]


def kernel(x, coeffs, importance):
    B, T, D = x.shape
    keep = max(1, int(D * _KEEP_RATIO))
    deg = coeffs.shape[0]
    n = B * T
    nsteps1 = n // _ROWS1
    nsteps2 = n // _ROWS2
    xf = x.reshape(n, D)

    mask = _make_sc_select(keep, D)(importance).reshape(1, D)

    colsum = pl.pallas_call(
        _tc_sumsq_kernel,
        grid=(nsteps1,),
        in_specs=[pl.BlockSpec((_ROWS1, D), lambda i: (i, 0))],
        out_specs=pl.BlockSpec((1, D), lambda i: (0, 0)),
        out_shape=jax.ShapeDtypeStruct((1, D), jnp.float32),
    )(xf)

    y = pl.pallas_call(
        _tc_out_kernel,
        grid=(nsteps2,),
        in_specs=[
            pl.BlockSpec((1, D), lambda i: (0, 0)),
            pl.BlockSpec((1, D), lambda i: (0, 0)),
            pl.BlockSpec((_ROWS2, D), lambda i: (i, 0)),
            pl.BlockSpec(memory_space=pltpu.SMEM),
        ],
        out_specs=pl.BlockSpec((_ROWS2, D), lambda i: (i, 0)),
        out_shape=jax.ShapeDtypeStruct((n, D), jnp.float32),
        scratch_shapes=[
            pltpu.VMEM((deg, D), jnp.float32),
            pltpu.SMEM((1, 1), jnp.float32),
        ],
    )(mask, colsum, xf, coeffs.reshape(1, deg))

    return y.reshape(B, T, D)
